# baseline (device time: 6622 ns/iter reference)
import jax
import jax.numpy as jnp
from jax import lax
from jax.experimental import pallas as pl
from jax.experimental.pallas import tpu as pltpu

N_DEV = 4


def kernel(x):
    m_per, n = x.shape

    def body(x_hbm, out_ref, x_vmem, copy_sem, comm_ref, send_sems, recv_sems):
        my_pos = lax.axis_index("i")

        barrier_sem = pltpu.get_barrier_semaphore()
        for d in range(1, N_DEV):
            pl.semaphore_signal(
                barrier_sem,
                inc=1,
                device_id=(lax.rem(my_pos + d, N_DEV),),
                device_id_type=pl.DeviceIdType.MESH,
            )

        copy = pltpu.make_async_copy(x_hbm, x_vmem, copy_sem)
        copy.start()
        copy.wait()

        comm_ref[0, :, :] = jnp.max(x_vmem[:, :], axis=0, keepdims=True)

        pl.semaphore_wait(barrier_sem, N_DEV - 1)

        rdmas = {}
        for d in (2, 1, 3):
            target = lax.rem(my_pos + d, N_DEV)
            rdma = pltpu.make_async_remote_copy(
                src_ref=comm_ref.at[0],
                dst_ref=comm_ref.at[d],
                send_sem=send_sems.at[d - 1],
                recv_sem=recv_sems.at[d - 1],
                device_id=(target,),
                device_id_type=pl.DeviceIdType.MESH,
            )
            rdma.start()
            rdmas[d] = rdma

        rdmas[1].wait_recv()
        rdmas[3].wait_recv()
        near = jnp.maximum(
            jnp.maximum(comm_ref[0, :, :], comm_ref[1, :, :]), comm_ref[3, :, :]
        )
        rdmas[2].wait_recv()
        out_ref[:, :] = jnp.maximum(near, comm_ref[2, :, :])

        for d in (1, 2, 3):
            rdmas[d].wait_send()

    return pl.pallas_call(
        body,
        out_shape=jax.ShapeDtypeStruct((1, n), x.dtype),
        in_specs=[pl.BlockSpec(memory_space=pl.ANY)],
        out_specs=pl.BlockSpec(memory_space=pltpu.VMEM),
        scratch_shapes=[
            pltpu.VMEM((m_per, n), x.dtype),
            pltpu.SemaphoreType.DMA,
            pltpu.VMEM((N_DEV, 1, n), x.dtype),
            pltpu.SemaphoreType.DMA((N_DEV - 1,)),
            pltpu.SemaphoreType.DMA((N_DEV - 1,)),
        ],
        compiler_params=pltpu.CompilerParams(collective_id=0),
    )(x)


# device time: 6554 ns/iter; 1.0104x vs baseline; 1.0104x over previous
import jax
import jax.numpy as jnp
from jax import lax
from jax.experimental import pallas as pl
from jax.experimental.pallas import tpu as pltpu

N_DEV = 4


def kernel(x):
    m_per, n = x.shape

    def body(x_ref, out_ref, comm_ref, send_sems, recv_sems):
        my_pos = lax.axis_index("i")

        barrier_sem = pltpu.get_barrier_semaphore()
        for d in range(1, N_DEV):
            pl.semaphore_signal(
                barrier_sem,
                inc=1,
                device_id=(lax.rem(my_pos + d, N_DEV),),
                device_id_type=pl.DeviceIdType.MESH,
            )

        comm_ref[0, :, :] = jnp.max(x_ref[:, :], axis=0, keepdims=True)

        pl.semaphore_wait(barrier_sem, N_DEV - 1)

        rdmas = {}
        for d in (2, 1, 3):
            target = lax.rem(my_pos + d, N_DEV)
            rdma = pltpu.make_async_remote_copy(
                src_ref=comm_ref.at[0],
                dst_ref=comm_ref.at[d],
                send_sem=send_sems.at[d - 1],
                recv_sem=recv_sems.at[d - 1],
                device_id=(target,),
                device_id_type=pl.DeviceIdType.MESH,
            )
            rdma.start()
            rdmas[d] = rdma

        rdmas[1].wait_recv()
        rdmas[3].wait_recv()
        near = jnp.maximum(
            jnp.maximum(comm_ref[0, :, :], comm_ref[1, :, :]), comm_ref[3, :, :]
        )
        rdmas[2].wait_recv()
        out_ref[:, :] = jnp.maximum(near, comm_ref[2, :, :])

        for d in (1, 2, 3):
            rdmas[d].wait_send()

    return pl.pallas_call(
        body,
        out_shape=jax.ShapeDtypeStruct((1, n), x.dtype),
        in_specs=[pl.BlockSpec(memory_space=pltpu.VMEM)],
        out_specs=pl.BlockSpec(memory_space=pltpu.VMEM),
        scratch_shapes=[
            pltpu.VMEM((N_DEV, 1, n), x.dtype),
            pltpu.SemaphoreType.DMA((N_DEV - 1,)),
            pltpu.SemaphoreType.DMA((N_DEV - 1,)),
        ],
        compiler_params=pltpu.CompilerParams(collective_id=0),
    )(x)
